# Initial kernel scaffold; baseline (speedup 1.0000x reference)
import functools

import jax
import jax.numpy as jnp
from jax import lax
from jax.experimental import pallas as pl
from jax.experimental.pallas import tpu as pltpu
from jax.experimental.pallas import tpu_sc as plsc

N = 10000
E = 320000
NC, NS = 2, 16
NW = NC * NS
CH = 128
CHUNKS = 80
EPW = CH * CHUNKS
RPW = 640
NP = NS * RPW
DUMMY = N
RB = 1024


def _mesh():
    return plsc.VectorSubcoreMesh(core_axis_name="c", subcore_axis_name="s")


def _sc_scatter(D):
    @functools.partial(
        pl.kernel,
        out_type=jax.ShapeDtypeStruct((NC, NP, D), jnp.float32),
        mesh=_mesh(),
        scratch_types=[
            pltpu.VMEM((CHUNKS, CH), jnp.int32),
            pltpu.VMEM((CHUNKS, CH), jnp.int32),
            pltpu.VMEM((CH, D), jnp.float32),
            pltpu.VMEM((CH, D), jnp.float32),
            pltpu.VMEM_SHARED((NP, D), jnp.float32),
            pltpu.SemaphoreType.DMA,
            pltpu.SemaphoreType.DMA,
        ],
    )
    def k(g_hbm, srcw_hbm, dstw_hbm, zrow_hbm, out_hbm,
          src_v, dst_v, buf0, buf1, acc, sem0, sem1):
        c = lax.axis_index("c")
        s = lax.axis_index("s")
        w = s * NC + c
        pltpu.sync_copy(srcw_hbm.at[pl.ds(w * CHUNKS, CHUNKS)], src_v)
        pltpu.sync_copy(dstw_hbm.at[pl.ds(w * CHUNKS, CHUNKS)], dst_v)
        pltpu.sync_copy(zrow_hbm, buf0)

        @pl.loop(0, RPW // CH)
        def _zero(i):
            pltpu.sync_copy(buf0, acc.at[pl.ds(s * RPW + i * CH, CH)])

        plsc.subcore_barrier()

        bufs = (buf0, buf1)
        sems = (sem0, sem1)
        for b in range(2):
            pltpu.async_copy(g_hbm.at[src_v.at[b]], bufs[b], sems[b])

        @pl.loop(0, CHUNKS, step=2)
        def _chunk(j):
            for b in range(2):
                cur = j + b
                pltpu.make_async_copy(g_hbm.at[src_v.at[cur]], bufs[b],
                                      sems[b]).wait()
                pltpu.sync_copy(bufs[b], acc.at[dst_v.at[cur]], add=True)
                nxt = cur + 2

                @pl.when(nxt < CHUNKS)
                def _():
                    pltpu.async_copy(g_hbm.at[src_v.at[nxt]], bufs[b], sems[b])

        plsc.subcore_barrier()
        pltpu.sync_copy(acc.at[pl.ds(s * RPW, RPW)],
                        out_hbm.at[c, pl.ds(s * RPW, RPW)])

    return k


def _sc_degree():
    @functools.partial(
        pl.kernel,
        out_type=jax.ShapeDtypeStruct((NC, NP, 16), jnp.float32),
        mesh=_mesh(),
        scratch_types=[
            pltpu.VMEM((CHUNKS, CH), jnp.int32),
            pltpu.VMEM((CH, 16), jnp.float32),
            pltpu.VMEM((CH, 16), jnp.float32),
            pltpu.VMEM_SHARED((NP, 16), jnp.float32),
        ],
    )
    def k(dstw_hbm, ones_hbm, zrow_hbm, out_hbm, dst_v, ones_v, zbuf, acc):
        c = lax.axis_index("c")
        s = lax.axis_index("s")
        w = s * NC + c
        pltpu.sync_copy(dstw_hbm.at[pl.ds(w * CHUNKS, CHUNKS)], dst_v)
        pltpu.sync_copy(zrow_hbm, zbuf)

        @pl.loop(0, RPW // CH)
        def _zero(i):
            pltpu.sync_copy(zbuf, acc.at[pl.ds(s * RPW + i * CH, CH)])

        pltpu.sync_copy(ones_hbm, ones_v)
        plsc.subcore_barrier()

        @pl.loop(0, CHUNKS)
        def _chunk(j):
            pltpu.sync_copy(ones_v, acc.at[dst_v.at[j]], add=True)

        plsc.subcore_barrier()
        pltpu.sync_copy(acc.at[pl.ds(s * RPW, RPW)],
                        out_hbm.at[c, pl.ds(s * RPW, RPW)])

    return k


def _dinv(deg_ref):
    deg = deg_ref[0, :, 0:1] + deg_ref[1, :, 0:1] + 1.0
    return lax.rsqrt(deg)


def _spec_rows(D):
    return pl.BlockSpec((RB, D), lambda i: (i, 0))


def _spec_parts(D):
    return pl.BlockSpec((NC, RB, D), lambda i: (0, i, 0))


def _spec_full(shape):
    return pl.BlockSpec(shape, lambda i: tuple(0 for _ in shape))


def _tc_call(body, in_specs, out_dim):
    return pl.pallas_call(
        body,
        grid=(NP // RB,),
        in_specs=in_specs,
        out_specs=_spec_rows(out_dim),
        out_shape=jax.ShapeDtypeStruct((NP, out_dim), jnp.float32),
    )


def _t_scale_mm(degp, z, W):
    di, do = W.shape

    def body(deg_ref, z_ref, w_ref, o_ref):
        o_ref[...] = _dinv(deg_ref) * jnp.dot(
            z_ref[...], w_ref[...], preferred_element_type=jnp.float32)

    return _tc_call(body, [_spec_parts(16), _spec_rows(di), _spec_full((di, do))],
                    do)(degp, z, W)


def _t_comb_mm(degp, S, g, b, W):
    di, do = W.shape

    def body(deg_ref, s_ref, g_ref, b_ref, w_ref, o_ref):
        dinv = _dinv(deg_ref)
        z = jnp.maximum(dinv * (s_ref[0] + s_ref[1] + g_ref[...]) + b_ref[...],
                        0.0)
        o_ref[...] = dinv * jnp.dot(z, w_ref[...],
                                    preferred_element_type=jnp.float32)

    return _tc_call(body, [_spec_parts(16), _spec_parts(di), _spec_rows(di),
                           _spec_full((1, di)), _spec_full((di, do))],
                    do)(degp, S, g, b.reshape(1, di), W)


def _t_comb_scale(degp, S, g, b):
    d = g.shape[1]

    def body(deg_ref, s_ref, g_ref, b_ref, o_ref):
        dinv = _dinv(deg_ref)
        z = jnp.maximum(dinv * (s_ref[0] + s_ref[1] + g_ref[...]) + b_ref[...],
                        0.0)
        o_ref[...] = dinv * z

    return _tc_call(body, [_spec_parts(16), _spec_parts(d), _spec_rows(d),
                           _spec_full((1, d))], d)(degp, S, g, b.reshape(1, d))


def _t_mm_post(degp, S, u, W, b):
    di, do = W.shape

    def body(deg_ref, s_ref, u_ref, w_ref, b_ref, o_ref):
        dinv = _dinv(deg_ref)
        m = dinv * (s_ref[0] + s_ref[1] + u_ref[...])
        z = jnp.maximum(jnp.dot(m, w_ref[...],
                                preferred_element_type=jnp.float32) + b_ref[...],
                        0.0)
        o_ref[...] = dinv * z

    return _tc_call(body, [_spec_parts(16), _spec_parts(di), _spec_rows(di),
                           _spec_full((di, do)), _spec_full((1, do))],
                    do)(degp, S, u, W, b.reshape(1, do))


def _t_mm2_post(degp, S, u, W, b, W2):
    di, dm = W.shape
    do = W2.shape[1]

    def body(deg_ref, s_ref, u_ref, w_ref, b_ref, w2_ref, o_ref):
        dinv = _dinv(deg_ref)
        m = dinv * (s_ref[0] + s_ref[1] + u_ref[...])
        z = jnp.maximum(jnp.dot(m, w_ref[...],
                                preferred_element_type=jnp.float32) + b_ref[...],
                        0.0)
        o_ref[...] = dinv * jnp.dot(z, w2_ref[...],
                                    preferred_element_type=jnp.float32)

    return _tc_call(body, [_spec_parts(16), _spec_parts(di), _spec_rows(di),
                           _spec_full((di, dm)), _spec_full((1, dm)),
                           _spec_full((dm, do))],
                    do)(degp, S, u, W, b.reshape(1, dm), W2)


def _t_final(degp, S, g, b):
    d = g.shape[1]

    def body(deg_ref, s_ref, g_ref, b_ref, o_ref):
        dinv = _dinv(deg_ref)
        o_ref[...] = dinv * (s_ref[0] + s_ref[1] + g_ref[...]) + b_ref[...]

    return _tc_call(body, [_spec_parts(16), _spec_parts(d), _spec_rows(d),
                           _spec_full((1, d))], d)(degp, S, g, b.reshape(1, d))


def kernel(x, edge_index, W1, b1, W2, b2, W3, b3, Wu3, bu3, Wu4, bu4, Wu5, bu5):
    src = edge_index[0].astype(jnp.int32)
    dst = edge_index[1].astype(jnp.int32)
    pad = NW * EPW - E
    fill = jnp.full((pad,), DUMMY, jnp.int32)
    srcw = jnp.concatenate([src, fill]).reshape(NW * CHUNKS, CH)
    dstw = jnp.concatenate([dst, fill]).reshape(NW * CHUNKS, CH)
    x_p = jnp.pad(x, ((0, NP - N), (0, 0)))

    z16 = jnp.zeros((CH, 16), jnp.float32)
    ones16 = jnp.ones((CH, 16), jnp.float32)
    z32 = jnp.zeros((CH, 32), jnp.float32)
    z64 = jnp.zeros((CH, 64), jnp.float32)
    z128 = jnp.zeros((CH, 128), jnp.float32)

    degp = _sc_degree()(dstw, ones16, z16)

    sc32 = _sc_scatter(32)
    sc64 = _sc_scatter(64)
    sc128 = _sc_scatter(128)

    g1 = _t_scale_mm(degp, x_p, W1)
    S1 = sc128(g1, srcw, dstw, z128)
    g2 = _t_comb_mm(degp, S1, g1, b1, W2)
    S2 = sc64(g2, srcw, dstw, z64)
    g3 = _t_comb_mm(degp, S2, g2, b2, W3)
    S3 = sc32(g3, srcw, dstw, z32)
    u4 = _t_comb_scale(degp, S3, g3, b3)
    S4 = sc32(u4, srcw, dstw, z32)
    u5 = _t_mm_post(degp, S4, u4, Wu3, bu3)
    S5 = sc64(u5, srcw, dstw, z64)
    g6 = _t_mm2_post(degp, S5, u5, Wu4, bu4, Wu5)
    S6 = sc128(g6, srcw, dstw, z128)
    outp = _t_final(degp, S6, g6, bu5)
    return outp[:N]


# speedup vs baseline: 8.2390x; 8.2390x over previous
import functools

import jax
import jax.numpy as jnp
from jax import lax
from jax.experimental import pallas as pl
from jax.experimental.pallas import tpu as pltpu
from jax.experimental.pallas import tpu_sc as plsc

N = 10000
E = 320000
NC, NS = 2, 16
NW = NC * NS
CH = 128
CHUNKS = 80
EPW = CH * CHUNKS
RPW = 640
NP = NS * RPW
DUMMY = N
IBLK = 16
RB = 1024


def _mesh():
    return plsc.VectorSubcoreMesh(core_axis_name="c", subcore_axis_name="s")


def _sc_scatter(D, ch):
    @functools.partial(
        pl.kernel,
        out_type=jax.ShapeDtypeStruct((NC, NP, D), jnp.float32),
        mesh=_mesh(),
        compiler_params=pltpu.CompilerParams(use_tc_tiling_on_sc=False),
        scratch_types=[
            pltpu.VMEM_SHARED((NP, D), jnp.float32),
        ],
    )
    def k(g_hbm, srcw_hbm, dstw_hbm, zrow_hbm, out_hbm, acc):
        pl.run_scoped(
            functools.partial(_scatter_body, ch, g_hbm, srcw_hbm, dstw_hbm,
                              zrow_hbm, out_hbm, acc),
            pltpu.VMEM((IBLK, ch), jnp.int32),
            pltpu.VMEM((IBLK, ch), jnp.int32),
            pltpu.VMEM((ch, D), jnp.float32),
            pltpu.VMEM((ch, D), jnp.float32),
            pltpu.SemaphoreType.DMA,
            pltpu.SemaphoreType.DMA,
        )

    return k


def _scatter_body(ch, g_hbm, srcw_hbm, dstw_hbm, zrow_hbm, out_hbm, acc,
                  src_v, dst_v, buf0, buf1, sem0, sem1):
        chunks = EPW // ch
        c = lax.axis_index("c")
        s = lax.axis_index("s")
        w = s * NC + c
        pltpu.sync_copy(zrow_hbm, buf0)

        @pl.loop(0, RPW // ch)
        def _zero(i):
            pltpu.sync_copy(buf0, acc.at[pl.ds(s * RPW + i * ch, ch)])

        plsc.subcore_barrier()

        bufs = (buf0, buf1)
        sems = (sem0, sem1)

        @pl.loop(0, chunks // IBLK)
        def _block(kb):
            base = w * chunks + kb * IBLK
            pltpu.sync_copy(srcw_hbm.at[pl.ds(base, IBLK)], src_v)
            pltpu.sync_copy(dstw_hbm.at[pl.ds(base, IBLK)], dst_v)
            for b in range(2):
                pltpu.async_copy(g_hbm.at[src_v.at[b]], bufs[b], sems[b])

            @pl.loop(0, IBLK, step=2)
            def _chunk(j):
                for b in range(2):
                    cur = j + b
                    pltpu.make_async_copy(g_hbm.at[src_v.at[cur]], bufs[b],
                                          sems[b]).wait()
                    pltpu.sync_copy(bufs[b], acc.at[dst_v.at[cur]], add=True)
                    nxt = cur + 2

                    @pl.when(nxt < IBLK)
                    def _():
                        pltpu.async_copy(g_hbm.at[src_v.at[nxt]], bufs[b],
                                         sems[b])

        plsc.subcore_barrier()

        @pl.loop(0, RPW // ch)
        def _wb(i):
            pltpu.sync_copy(acc.at[pl.ds(s * RPW + i * ch, ch)],
                            out_hbm.at[c, pl.ds(s * RPW + i * ch, ch)])


def _dinv(deg_ref):
    deg = deg_ref[0, :, 0:1] + deg_ref[1, :, 0:1] + 1.0
    return lax.rsqrt(deg)


def _spec_rows(D):
    return pl.BlockSpec((RB, D), lambda i: (i, 0))


def _spec_parts(D):
    return pl.BlockSpec((NC, RB, D), lambda i: (0, i, 0))


def _spec_full(shape):
    return pl.BlockSpec(shape, lambda i: tuple(0 for _ in shape))


def _tc_call(body, in_specs, out_dim):
    return pl.pallas_call(
        body,
        grid=(NP // RB,),
        in_specs=in_specs,
        out_specs=_spec_rows(out_dim),
        out_shape=jax.ShapeDtypeStruct((NP, out_dim), jnp.float32),
    )


def _t_scale_mm(degp, z, W):
    di, do = W.shape

    def body(deg_ref, z_ref, w_ref, o_ref):
        o_ref[...] = _dinv(deg_ref) * jnp.dot(
            z_ref[...], w_ref[...], preferred_element_type=jnp.float32)

    return _tc_call(body, [_spec_parts(32), _spec_rows(di), _spec_full((di, do))],
                    do)(degp, z, W)


def _t_comb_mm(degp, S, g, b, W):
    di, do = W.shape

    def body(deg_ref, s_ref, g_ref, b_ref, w_ref, o_ref):
        dinv = _dinv(deg_ref)
        z = jnp.maximum(dinv * (s_ref[0] + s_ref[1] + g_ref[...]) + b_ref[...],
                        0.0)
        o_ref[...] = dinv * jnp.dot(z, w_ref[...],
                                    preferred_element_type=jnp.float32)

    return _tc_call(body, [_spec_parts(32), _spec_parts(di), _spec_rows(di),
                           _spec_full((1, di)), _spec_full((di, do))],
                    do)(degp, S, g, b.reshape(1, di), W)


def _t_comb_scale(degp, S, g, b):
    d = g.shape[1]

    def body(deg_ref, s_ref, g_ref, b_ref, o_ref):
        dinv = _dinv(deg_ref)
        z = jnp.maximum(dinv * (s_ref[0] + s_ref[1] + g_ref[...]) + b_ref[...],
                        0.0)
        o_ref[...] = dinv * z

    return _tc_call(body, [_spec_parts(32), _spec_parts(d), _spec_rows(d),
                           _spec_full((1, d))], d)(degp, S, g, b.reshape(1, d))


def _t_mm_post(degp, S, u, W, b):
    di, do = W.shape

    def body(deg_ref, s_ref, u_ref, w_ref, b_ref, o_ref):
        dinv = _dinv(deg_ref)
        m = dinv * (s_ref[0] + s_ref[1] + u_ref[...])
        z = jnp.maximum(jnp.dot(m, w_ref[...],
                                preferred_element_type=jnp.float32) + b_ref[...],
                        0.0)
        o_ref[...] = dinv * z

    return _tc_call(body, [_spec_parts(32), _spec_parts(di), _spec_rows(di),
                           _spec_full((di, do)), _spec_full((1, do))],
                    do)(degp, S, u, W, b.reshape(1, do))


def _t_mm2_post(degp, S, u, W, b, W2):
    di, dm = W.shape
    do = W2.shape[1]

    def body(deg_ref, s_ref, u_ref, w_ref, b_ref, w2_ref, o_ref):
        dinv = _dinv(deg_ref)
        m = dinv * (s_ref[0] + s_ref[1] + u_ref[...])
        z = jnp.maximum(jnp.dot(m, w_ref[...],
                                preferred_element_type=jnp.float32) + b_ref[...],
                        0.0)
        o_ref[...] = dinv * jnp.dot(z, w2_ref[...],
                                    preferred_element_type=jnp.float32)

    return _tc_call(body, [_spec_parts(32), _spec_parts(di), _spec_rows(di),
                           _spec_full((di, dm)), _spec_full((1, dm)),
                           _spec_full((dm, do))],
                    do)(degp, S, u, W, b.reshape(1, dm), W2)


def _t_final(degp, S, g, b):
    d = g.shape[1]

    def body(deg_ref, s_ref, g_ref, b_ref, o_ref):
        dinv = _dinv(deg_ref)
        o_ref[...] = dinv * (s_ref[0] + s_ref[1] + g_ref[...]) + b_ref[...]

    return _tc_call(body, [_spec_parts(32), _spec_parts(d), _spec_rows(d),
                           _spec_full((1, d))], d)(degp, S, g, b.reshape(1, d))


def kernel(x, edge_index, W1, b1, W2, b2, W3, b3, Wu3, bu3, Wu4, bu4, Wu5, bu5):
    src = edge_index[0].astype(jnp.int32)
    dst = edge_index[1].astype(jnp.int32)
    pad = NW * EPW - E
    fill = jnp.full((pad,), DUMMY, jnp.int32)
    src_p = jnp.concatenate([src, fill])
    dst_p = jnp.concatenate([dst, fill])
    srcw = src_p.reshape(NW * CHUNKS, CH)
    dstw = dst_p.reshape(NW * CHUNKS, CH)
    x_p = jnp.pad(x, ((0, NP - N), (0, 0)))

    z32 = jnp.zeros((CH, 32), jnp.float32)
    z128 = jnp.zeros((CH, 128), jnp.float32)

    sc32 = _sc_scatter(32, CH)
    sc128 = _sc_scatter(128, CH)

    ones_tab = jnp.ones((NP, 32), jnp.float32)
    degp = sc32(ones_tab, srcw, dstw, z32)

    def sc64(g):
        a = sc32(g[:, :32], srcw, dstw, z32)
        b = sc32(g[:, 32:], srcw, dstw, z32)
        return jnp.concatenate([a, b], axis=2)

    g1 = _t_scale_mm(degp, x_p, W1)
    S1 = sc128(g1, srcw, dstw, z128)
    g2 = _t_comb_mm(degp, S1, g1, b1, W2)
    S2 = sc64(g2)
    g3 = _t_comb_mm(degp, S2, g2, b2, W3)
    S3 = sc32(g3, srcw, dstw, z32)
    u4 = _t_comb_scale(degp, S3, g3, b3)
    S4 = sc32(u4, srcw, dstw, z32)
    u5 = _t_mm_post(degp, S4, u4, Wu3, bu3)
    S5 = sc64(u5)
    g6 = _t_mm2_post(degp, S5, u5, Wu4, bu4, Wu5)
    S6 = sc128(g6, srcw, dstw, z128)
    outp = _t_final(degp, S6, g6, bu5)
    return outp[:N]
